# baseline (speedup 1.0000x reference)
import functools

import jax
import jax.numpy as jnp
from jax import lax
from jax.experimental import pallas as pl
from jax.experimental.pallas import tpu as pltpu
from jax.experimental.pallas import tpu_sc as plsc

N = 10000
E = 320000
D = 128
MSG = 16
PPAD = 16
MROW = 32

NC = 2
NS = 16
NW = NC * NS
PER_W = E // NW
G_CH = 400
S_CH = 1000

def _silu(v):
    half = jnp.asarray(0.5, v.dtype)
    return v * (half * jnp.tanh(v * half) + half)


@functools.lru_cache(maxsize=None)
def _get_gather_kernel():
    mesh = plsc.VectorSubcoreMesh(core_axis_name="c", subcore_axis_name="s")

    @functools.partial(
        pl.kernel,
        out_type=(
            jax.ShapeDtypeStruct((E, D), jnp.int32),
            jax.ShapeDtypeStruct((E, 2 * PPAD), jnp.float32),
        ),
        mesh=mesh,
        scratch_types=[
            pltpu.VMEM((G_CH,), jnp.int32),
            pltpu.VMEM((G_CH,), jnp.int32),
            pltpu.VMEM((G_CH, D // 2), jnp.int32),
            pltpu.VMEM((G_CH, D // 2), jnp.int32),
            pltpu.VMEM((G_CH, PPAD), jnp.float32),
            pltpu.VMEM((G_CH, PPAD), jnp.float32),
            pltpu.SemaphoreType.DMA,
        ],
        compiler_params=pltpu.CompilerParams(use_tc_tiling_on_sc=False),
    )
    def _gather_kernel(tx, tp, dst, src, gx, gp,
                       idx_d, idx_s, xb_d, xb_s, pb_d, pb_s, sem):
        w = lax.axis_index("s") * NC + lax.axis_index("c")
        base = w * PER_W

        @pl.loop(0, PER_W, step=G_CH)
        def _(off):
            e0 = base + off
            pltpu.sync_copy(dst.at[pl.ds(e0, G_CH)], idx_d)
            pltpu.sync_copy(src.at[pl.ds(e0, G_CH)], idx_s)
            c1 = pltpu.async_copy(tx.at[idx_d], xb_d, sem)
            c2 = pltpu.async_copy(tx.at[idx_s], xb_s, sem)
            c3 = pltpu.async_copy(tp.at[idx_d], pb_d, sem)
            c4 = pltpu.async_copy(tp.at[idx_s], pb_s, sem)
            c1.wait()
            c2.wait()
            c3.wait()
            c4.wait()
            pltpu.sync_copy(xb_d, gx.at[pl.ds(e0, G_CH), pl.ds(0, D // 2)])
            pltpu.sync_copy(xb_s, gx.at[pl.ds(e0, G_CH), pl.ds(D // 2, D // 2)])
            pltpu.sync_copy(pb_d, gp.at[pl.ds(e0, G_CH), pl.ds(0, PPAD)])
            pltpu.sync_copy(pb_s, gp.at[pl.ds(e0, G_CH), pl.ds(PPAD, PPAD)])

    return _gather_kernel


EB = 2000


def _edge_body(gx, gp, w1ab, w1c16, b1, w2, b2, w5, b5, w6, b6, out):
    rel = gp[:, 0:PPAD] - gp[:, PPAD:2 * PPAD]
    rel2 = rel * rel
    acc = lax.dot_general(gx[...], w1ab[...], (((1,), (0,)), ((), ())),
                          preferred_element_type=jnp.float32)
    accd = lax.dot_general(rel2, w1c16[...], (((1,), (0,)), ((), ())),
                           preferred_element_type=jnp.float32)
    hpre = (acc + accd + b1[...]).astype(jnp.bfloat16)
    h = _silu(hpre)
    mpre = lax.dot_general(h, w2[...], (((1,), (0,)), ((), ())),
                           preferred_element_type=jnp.float32) + b2[...]
    msg = _silu(mpre)
    t = _silu(jnp.dot(msg, w5[...], preferred_element_type=jnp.float32)
              + b5[...])
    cw = jnp.dot(t, w6[...], preferred_element_type=jnp.float32) + b6[...]
    out[:, 0:MSG] = msg
    out[:, MSG:MROW] = rel * cw


def _edge_mlp(gx, gp, w1ab, w1c16, b1, w2, b2, w5, b5, w6, b6):
    ein = 2 * D + 1
    const = lambda shape: pl.BlockSpec(shape, lambda i: (0, 0))
    return pl.pallas_call(
        _edge_body,
        grid=(E // EB,),
        in_specs=[
            pl.BlockSpec((EB, 2 * D), lambda i: (i, 0)),
            pl.BlockSpec((EB, 2 * PPAD), lambda i: (i, 0)),
            const((2 * D, ein)),
            const((PPAD, ein)),
            const((1, ein)),
            const((ein, MSG)),
            const((1, MSG)),
            const((MSG, 2 * MSG)),
            const((1, 2 * MSG)),
            const((2 * MSG, 1)),
            const((1, 1)),
        ],
        out_specs=pl.BlockSpec((EB, MROW), lambda i: (i, 0)),
        out_shape=jax.ShapeDtypeStruct((E, MROW), jnp.float32),
    )(gx, gp, w1ab, w1c16, b1, w2, b2, w5, b5, w6, b6)


@functools.lru_cache(maxsize=None)
def _get_scatter_kernel():
    mesh = plsc.VectorSubcoreMesh(core_axis_name="c", subcore_axis_name="s")

    @functools.partial(
        pl.kernel,
        out_type=jax.ShapeDtypeStruct((NC, N, MROW), jnp.float32),
        mesh=mesh,
        scratch_types=[
            pltpu.VMEM((S_CH,), jnp.int32),
            pltpu.VMEM((S_CH, MROW), jnp.float32),
            pltpu.VMEM_SHARED((N, MROW), jnp.float32),
            pltpu.SemaphoreType.DMA,
        ],
        compiler_params=pltpu.CompilerParams(use_tc_tiling_on_sc=False),
    )
    def _scatter_kernel(m, dst, zeros, out, idx_v, rows_v, accum, sem):
        c = lax.axis_index("c")
        s = lax.axis_index("s")
        w = s * NC + c
        base = w * PER_W

        @pl.when(s == 0)
        def _():
            pltpu.sync_copy(zeros, accum)

        plsc.subcore_barrier()

        @pl.loop(0, PER_W, step=S_CH)
        def _(off):
            e0 = base + off
            pltpu.sync_copy(dst.at[pl.ds(e0, S_CH)], idx_v)
            pltpu.sync_copy(m.at[pl.ds(e0, S_CH)], rows_v)
            pltpu.sync_copy(rows_v, accum.at[idx_v], add=True)

        plsc.subcore_barrier()

        @pl.when(s == 0)
        def _():
            pltpu.sync_copy(accum, out.at[c])

    return _scatter_kernel


NB = 2000


def _node_body(x, pos, p0, p1, w3a, w3b, b3, w4, b4, out_x, out_pos):
    agg = p0[...] + p1[...]
    am = agg[:, 0:MSG]
    ap = agg[:, MSG:MSG + 3]
    h1 = (jnp.dot(x[...], w3a[...], preferred_element_type=jnp.float32)
          + jnp.dot(am, w3b[...], preferred_element_type=jnp.float32)
          + b3[...])
    out_x[...] = (jnp.dot(_silu(h1), w4[...],
                          preferred_element_type=jnp.float32) + b4[...])
    out_pos[...] = pos[...] + ap


def _node_mlp(x, pos, p0, p1, w3a, w3b, b3, w4, b4):
    const = lambda shape: pl.BlockSpec(shape, lambda i: (0, 0))
    return pl.pallas_call(
        _node_body,
        grid=(N // NB,),
        in_specs=[
            pl.BlockSpec((NB, D), lambda i: (i, 0)),
            pl.BlockSpec((NB, 3), lambda i: (i, 0)),
            pl.BlockSpec((NB, MROW), lambda i: (i, 0)),
            pl.BlockSpec((NB, MROW), lambda i: (i, 0)),
            const((D, D)),
            const((MSG, D)),
            const((1, D)),
            const((D, D)),
            const((1, D)),
        ],
        out_specs=[
            pl.BlockSpec((NB, D), lambda i: (i, 0)),
            pl.BlockSpec((NB, 3), lambda i: (i, 0)),
        ],
        out_shape=[
            jax.ShapeDtypeStruct((N, D), jnp.float32),
            jax.ShapeDtypeStruct((N, 3), jnp.float32),
        ],
    )(x, pos, p0, p1, w3a, w3b, b3, w4, b4)


def kernel(x, pos, edge_index, W1, b1, W2, b2, W3, b3, W4, b4, W5, b5, W6, b6):
    src = edge_index[0].astype(jnp.int32)
    dst = edge_index[1].astype(jnp.int32)

    tp = jnp.pad(pos, ((0, 0), (0, PPAD - 3)))
    txp = jax.lax.bitcast_convert_type(
        x.astype(jnp.bfloat16).reshape(N, D // 2, 2), jnp.int32)

    gxp, gp = _get_gather_kernel()(txp, tp, dst, src)
    gx = jax.lax.bitcast_convert_type(gxp, jnp.bfloat16).reshape(E, 2 * D)

    w1ab = W1[:2 * D].astype(jnp.bfloat16)
    w1c16 = jnp.tile(W1[2 * D:], (PPAD, 1))
    m = _edge_mlp(gx, gp, w1ab, w1c16, b1[None, :],
                  W2.astype(jnp.bfloat16), b2[None, :], W5, b5[None, :],
                  W6, b6[None, :])

    partials = _get_scatter_kernel()(m, dst, jnp.zeros((N, MROW), jnp.float32))

    out_x, out_pos = _node_mlp(x, pos, partials[0], partials[1],
                               W3[:D], W3[D:], b3[None, :], W4, b4[None, :])
    return (out_x, out_pos)


# speedup vs baseline: 2.1107x; 2.1107x over previous
import functools

import jax
import jax.numpy as jnp
from jax import lax
from jax.experimental import pallas as pl
from jax.experimental.pallas import tpu as pltpu
from jax.experimental.pallas import tpu_sc as plsc

N = 10000
E = 320000
D = 128
MSG = 16
PPAD = 16
MROW = 32

NC = 2
NS = 16
NW = NC * NS
PER_W = E // NW
G_CH = 400
S_CH = 1000

def _silu(v):
    half = jnp.asarray(0.5, v.dtype)
    return v * (half * jnp.tanh(v * half) + half)


@functools.lru_cache(maxsize=None)
def _get_gather_kernel():
    mesh = plsc.VectorSubcoreMesh(core_axis_name="c", subcore_axis_name="s")

    @functools.partial(
        pl.kernel,
        out_type=(
            jax.ShapeDtypeStruct((E, D), jnp.int32),
            jax.ShapeDtypeStruct((E, PPAD), jnp.float32),
            jax.ShapeDtypeStruct((E, PPAD), jnp.float32),
        ),
        mesh=mesh,
        scratch_types=[
            pltpu.VMEM((G_CH,), jnp.int32),
            pltpu.VMEM((G_CH,), jnp.int32),
            pltpu.VMEM((G_CH, D // 2), jnp.int32),
            pltpu.VMEM((G_CH, D // 2), jnp.int32),
            pltpu.VMEM((G_CH, PPAD), jnp.float32),
            pltpu.VMEM((G_CH, PPAD), jnp.float32),
            pltpu.SemaphoreType.DMA,
        ],
        compiler_params=pltpu.CompilerParams(use_tc_tiling_on_sc=False),
    )
    def _gather_kernel(tx, tp, dst, src, gx, gpd, gps,
                       idx_d, idx_s, xb_d, xb_s, pb_d, pb_s, sem):
        w = lax.axis_index("s") * NC + lax.axis_index("c")
        base = w * PER_W

        @pl.loop(0, PER_W, step=G_CH)
        def _(off):
            e0 = base + off
            pltpu.sync_copy(dst.at[pl.ds(e0, G_CH)], idx_d)
            pltpu.sync_copy(src.at[pl.ds(e0, G_CH)], idx_s)
            c1 = pltpu.async_copy(tx.at[idx_d], xb_d, sem)
            c2 = pltpu.async_copy(tx.at[idx_s], xb_s, sem)
            c3 = pltpu.async_copy(tp.at[idx_d], pb_d, sem)
            c4 = pltpu.async_copy(tp.at[idx_s], pb_s, sem)
            c1.wait()
            c2.wait()
            c3.wait()
            c4.wait()
            pltpu.sync_copy(xb_d, gx.at[pl.ds(e0, G_CH), pl.ds(0, D // 2)])
            pltpu.sync_copy(xb_s, gx.at[pl.ds(e0, G_CH), pl.ds(D // 2, D // 2)])
            pltpu.sync_copy(pb_d, gpd.at[pl.ds(e0, G_CH)])
            pltpu.sync_copy(pb_s, gps.at[pl.ds(e0, G_CH)])

    return _gather_kernel


EB = 2000


def _edge_body(gx, gpd, gps, w1ab, w1c16, b1, w2, b2, w5, b5, w6, b6, out):
    rel = gpd[...] - gps[...]
    rel2 = rel * rel
    w = gx[...]
    xlo = lax.bitcast_convert_type(w << 16, jnp.float32)
    xhi = lax.bitcast_convert_type(w & jnp.int32(-65536), jnp.float32)
    xcat = jnp.concatenate([xlo, xhi], axis=1)
    acc = lax.dot_general(xcat, w1ab[...], (((1,), (0,)), ((), ())),
                          preferred_element_type=jnp.float32)
    accd = lax.dot_general(rel2, w1c16[...], (((1,), (0,)), ((), ())),
                           preferred_element_type=jnp.float32)
    hpre = (acc + accd + b1[...]).astype(jnp.bfloat16)
    h = _silu(hpre)
    mpre = lax.dot_general(h, w2[...], (((1,), (0,)), ((), ())),
                           preferred_element_type=jnp.float32) + b2[...]
    msg = _silu(mpre)
    t = _silu(jnp.dot(msg, w5[...], preferred_element_type=jnp.float32)
              + b5[...])
    cw = jnp.dot(t, w6[...], preferred_element_type=jnp.float32) + b6[...]
    out[:, 0:MSG] = msg
    out[:, MSG:MROW] = rel * cw


def _edge_mlp(gx, gpd, gps, w1ab, w1c16, b1, w2, b2, w5, b5, w6, b6):
    ein = 2 * D + 1
    const = lambda shape: pl.BlockSpec(shape, lambda i: (0, 0))
    return pl.pallas_call(
        _edge_body,
        grid=(E // EB,),
        in_specs=[
            pl.BlockSpec((EB, D), lambda i: (i, 0)),
            pl.BlockSpec((EB, PPAD), lambda i: (i, 0)),
            pl.BlockSpec((EB, PPAD), lambda i: (i, 0)),
            const((2 * D, ein)),
            const((PPAD, ein)),
            const((1, ein)),
            const((ein, MSG)),
            const((1, MSG)),
            const((MSG, 2 * MSG)),
            const((1, 2 * MSG)),
            const((2 * MSG, 1)),
            const((1, 1)),
        ],
        out_specs=pl.BlockSpec((EB, MROW), lambda i: (i, 0)),
        out_shape=jax.ShapeDtypeStruct((E, MROW), jnp.float32),
    )(gx, gpd, gps, w1ab, w1c16, b1, w2, b2, w5, b5, w6, b6)


@functools.lru_cache(maxsize=None)
def _get_scatter_kernel():
    mesh = plsc.VectorSubcoreMesh(core_axis_name="c", subcore_axis_name="s")

    @functools.partial(
        pl.kernel,
        out_type=jax.ShapeDtypeStruct((NC, N, MROW), jnp.float32),
        mesh=mesh,
        scratch_types=[
            pltpu.VMEM((S_CH,), jnp.int32),
            pltpu.VMEM((S_CH, MROW), jnp.float32),
            pltpu.VMEM_SHARED((N, MROW), jnp.float32),
            pltpu.SemaphoreType.DMA,
        ],
        compiler_params=pltpu.CompilerParams(use_tc_tiling_on_sc=False),
    )
    def _scatter_kernel(m, dst, zeros, out, idx_v, rows_v, accum, sem):
        c = lax.axis_index("c")
        s = lax.axis_index("s")
        w = s * NC + c
        base = w * PER_W

        @pl.when(s == 0)
        def _():
            pltpu.sync_copy(zeros, accum)

        plsc.subcore_barrier()

        @pl.loop(0, PER_W, step=S_CH)
        def _(off):
            e0 = base + off
            pltpu.sync_copy(dst.at[pl.ds(e0, S_CH)], idx_v)
            pltpu.sync_copy(m.at[pl.ds(e0, S_CH)], rows_v)
            pltpu.sync_copy(rows_v, accum.at[idx_v], add=True)

        plsc.subcore_barrier()

        @pl.when(s == 0)
        def _():
            pltpu.sync_copy(accum, out.at[c])

    return _scatter_kernel


NB = 2000


def _node_body(x, pos, p0, p1, w3a, w3b, b3, w4, b4, out_x, out_pos):
    agg = p0[...] + p1[...]
    am = agg[:, 0:MSG]
    ap = agg[:, MSG:MSG + 3]
    h1 = (jnp.dot(x[...], w3a[...], preferred_element_type=jnp.float32)
          + jnp.dot(am, w3b[...], preferred_element_type=jnp.float32)
          + b3[...])
    out_x[...] = (jnp.dot(_silu(h1), w4[...],
                          preferred_element_type=jnp.float32) + b4[...])
    out_pos[...] = pos[...] + ap


def _node_mlp(x, pos, p0, p1, w3a, w3b, b3, w4, b4):
    const = lambda shape: pl.BlockSpec(shape, lambda i: (0, 0))
    return pl.pallas_call(
        _node_body,
        grid=(N // NB,),
        in_specs=[
            pl.BlockSpec((NB, D), lambda i: (i, 0)),
            pl.BlockSpec((NB, 3), lambda i: (i, 0)),
            pl.BlockSpec((NB, MROW), lambda i: (i, 0)),
            pl.BlockSpec((NB, MROW), lambda i: (i, 0)),
            const((D, D)),
            const((MSG, D)),
            const((1, D)),
            const((D, D)),
            const((1, D)),
        ],
        out_specs=[
            pl.BlockSpec((NB, D), lambda i: (i, 0)),
            pl.BlockSpec((NB, 3), lambda i: (i, 0)),
        ],
        out_shape=[
            jax.ShapeDtypeStruct((N, D), jnp.float32),
            jax.ShapeDtypeStruct((N, 3), jnp.float32),
        ],
    )(x, pos, p0, p1, w3a, w3b, b3, w4, b4)


def kernel(x, pos, edge_index, W1, b1, W2, b2, W3, b3, W4, b4, W5, b5, W6, b6):
    src = edge_index[0].astype(jnp.int32)
    dst = edge_index[1].astype(jnp.int32)

    tp = jnp.pad(pos, ((0, 0), (0, PPAD - 3)))
    xu = jax.lax.bitcast_convert_type(
        x.astype(jnp.bfloat16), jnp.uint16).astype(jnp.uint32)
    txp = jax.lax.bitcast_convert_type(
        xu[:, :D // 2] | (xu[:, D // 2:] << 16), jnp.int32)

    gxp, gpd, gps = _get_gather_kernel()(txp, tp, dst, src)

    H = D // 2
    w1ab = jnp.concatenate(
        [W1[0:H], W1[D:D + H], W1[H:D], W1[D + H:2 * D]])
    w1c16 = jnp.tile(W1[2 * D:], (PPAD, 1))
    m = _edge_mlp(gxp, gpd, gps, w1ab, w1c16, b1[None, :],
                  W2.astype(jnp.bfloat16), b2[None, :], W5, b5[None, :],
                  W6, b6[None, :])

    partials = _get_scatter_kernel()(m, dst, jnp.zeros((N, MROW), jnp.float32))

    out_x, out_pos = _node_mlp(x, pos, partials[0], partials[1],
                               W3[:D], W3[D:], b3[None, :], W4, b4[None, :])
    return (out_x, out_pos)


# speedup vs baseline: 2.4062x; 1.1400x over previous
import functools

import jax
import jax.numpy as jnp
from jax import lax
from jax.experimental import pallas as pl
from jax.experimental.pallas import tpu as pltpu
from jax.experimental.pallas import tpu_sc as plsc

N = 10000
E = 320000
D = 128
MSG = 16
PPAD = 16
MROW = 32

NC = 2
NS = 16
NW = NC * NS
PER_W = E // NW
G_CH = 400
NIT = PER_W // G_CH
S_CH = 1000

def _silu(v):
    half = jnp.asarray(0.5, v.dtype)
    return v * (half * jnp.tanh(v * half) + half)


@functools.lru_cache(maxsize=None)
def _get_gather_kernel():
    mesh = plsc.VectorSubcoreMesh(core_axis_name="c", subcore_axis_name="s")

    @functools.partial(
        pl.kernel,
        out_type=(
            jax.ShapeDtypeStruct((E, D), jnp.int32),
            jax.ShapeDtypeStruct((E, PPAD), jnp.float32),
            jax.ShapeDtypeStruct((E, PPAD), jnp.float32),
        ),
        mesh=mesh,
        scratch_types=[
            pltpu.VMEM((2, G_CH), jnp.int32),
            pltpu.VMEM((2, G_CH), jnp.int32),
            pltpu.VMEM((2, G_CH, D // 2), jnp.int32),
            pltpu.VMEM((2, G_CH, D // 2), jnp.int32),
            pltpu.VMEM((G_CH, PPAD), jnp.float32),
            pltpu.VMEM((G_CH, PPAD), jnp.float32),
            pltpu.SemaphoreType.DMA,
            pltpu.SemaphoreType.DMA,
            pltpu.SemaphoreType.DMA,
            pltpu.SemaphoreType.DMA,
        ],
        compiler_params=pltpu.CompilerParams(use_tc_tiling_on_sc=False),
    )
    def _gather_kernel(tx, tp, dst, src, gx, gpd, gps,
                       idx_d, idx_s, xb_d, xb_s, pb_d, pb_s,
                       sem_a, sem_b, sem_i, sem_p):
        w = lax.axis_index("s") * NC + lax.axis_index("c")
        base = w * PER_W
        gsem = (sem_a, sem_b)

        def load_idx(i, buf):
            e0 = base + i * G_CH
            pltpu.async_copy(dst.at[pl.ds(e0, G_CH)], idx_d.at[buf], sem_i)
            pltpu.async_copy(src.at[pl.ds(e0, G_CH)], idx_s.at[buf], sem_i)

        def wait_idx(buf):
            pltpu.make_async_copy(dst.at[pl.ds(0, G_CH)], idx_d.at[buf],
                                  sem_i).wait()
            pltpu.make_async_copy(src.at[pl.ds(0, G_CH)], idx_s.at[buf],
                                  sem_i).wait()

        def fire_x(buf):
            pltpu.async_copy(tx.at[idx_d.at[buf]], xb_d.at[buf], gsem[buf])
            pltpu.async_copy(tx.at[idx_s.at[buf]], xb_s.at[buf], gsem[buf])

        def wait_x(buf):
            pltpu.make_async_copy(tx.at[idx_d.at[buf]], xb_d.at[buf],
                                  gsem[buf]).wait()
            pltpu.make_async_copy(tx.at[idx_s.at[buf]], xb_s.at[buf],
                                  gsem[buf]).wait()

        def fire_pos(buf):
            pltpu.async_copy(tp.at[idx_d.at[buf]], pb_d, sem_p)
            pltpu.async_copy(tp.at[idx_s.at[buf]], pb_s, sem_p)

        def wait_pos():
            pltpu.make_async_copy(tp.at[idx_d.at[0]], pb_d, sem_p).wait()
            pltpu.make_async_copy(tp.at[idx_s.at[0]], pb_s, sem_p).wait()

        def wb_x(i, buf):
            e0 = base + i * G_CH
            pltpu.sync_copy(xb_d.at[buf],
                            gx.at[pl.ds(e0, G_CH), pl.ds(0, D // 2)])
            pltpu.sync_copy(xb_s.at[buf],
                            gx.at[pl.ds(e0, G_CH), pl.ds(D // 2, D // 2)])

        def wb_pos(i):
            e0 = base + i * G_CH
            pltpu.sync_copy(pb_d, gpd.at[pl.ds(e0, G_CH)])
            pltpu.sync_copy(pb_s, gps.at[pl.ds(e0, G_CH)])

        load_idx(0, 0)
        wait_idx(0)
        fire_x(0)
        fire_pos(0)
        load_idx(1, 1)

        @pl.loop(0, (NIT - 1) // 2)
        def _(k):
            i1 = 2 * k + 1
            wait_idx(1)
            fire_x(1)
            wait_x(0)
            wb_x(i1 - 1, 0)
            wait_pos()
            wb_pos(i1 - 1)
            fire_pos(1)

            @pl.when(i1 + 1 < NIT)
            def _():
                load_idx(i1 + 1, 0)
                wait_idx(0)
                fire_x(0)

            wait_x(1)
            wb_x(i1, 1)
            wait_pos()
            wb_pos(i1)

            @pl.when(i1 + 1 < NIT)
            def _():
                fire_pos(0)

            @pl.when(i1 + 2 < NIT)
            def _():
                load_idx(i1 + 2, 1)

        wait_x(0)
        wb_x(NIT - 1, 0)
        wait_pos()
        wb_pos(NIT - 1)

    return _gather_kernel


EB = 2000


def _edge_body(gx, gpd, gps, w1ab, w1c16, b1, w2, b2, w5, b5, w6, b6, out):
    rel = gpd[...] - gps[...]
    rel2 = rel * rel
    w = gx[...]
    xlo = lax.bitcast_convert_type(w << 16, jnp.float32)
    xhi = lax.bitcast_convert_type(w & jnp.int32(-65536), jnp.float32)
    xcat = jnp.concatenate([xlo, xhi], axis=1)
    acc = lax.dot_general(xcat, w1ab[...], (((1,), (0,)), ((), ())),
                          preferred_element_type=jnp.float32)
    accd = lax.dot_general(rel2, w1c16[...], (((1,), (0,)), ((), ())),
                           preferred_element_type=jnp.float32)
    hpre = (acc + accd + b1[...]).astype(jnp.bfloat16)
    h = _silu(hpre)
    mpre = lax.dot_general(h, w2[...], (((1,), (0,)), ((), ())),
                           preferred_element_type=jnp.float32) + b2[...]
    msg = _silu(mpre)
    t = _silu(jnp.dot(msg, w5[...], preferred_element_type=jnp.float32)
              + b5[...])
    cw = jnp.dot(t, w6[...], preferred_element_type=jnp.float32) + b6[...]
    out[:, 0:MSG] = msg
    out[:, MSG:MROW] = rel * cw


def _edge_mlp(gx, gpd, gps, w1ab, w1c16, b1, w2, b2, w5, b5, w6, b6):
    ein = 2 * D + 1
    const = lambda shape: pl.BlockSpec(shape, lambda i: (0, 0))
    return pl.pallas_call(
        _edge_body,
        grid=(E // EB,),
        in_specs=[
            pl.BlockSpec((EB, D), lambda i: (i, 0)),
            pl.BlockSpec((EB, PPAD), lambda i: (i, 0)),
            pl.BlockSpec((EB, PPAD), lambda i: (i, 0)),
            const((2 * D, ein)),
            const((PPAD, ein)),
            const((1, ein)),
            const((ein, MSG)),
            const((1, MSG)),
            const((MSG, 2 * MSG)),
            const((1, 2 * MSG)),
            const((2 * MSG, 1)),
            const((1, 1)),
        ],
        out_specs=pl.BlockSpec((EB, MROW), lambda i: (i, 0)),
        out_shape=jax.ShapeDtypeStruct((E, MROW), jnp.float32),
    )(gx, gpd, gps, w1ab, w1c16, b1, w2, b2, w5, b5, w6, b6)


@functools.lru_cache(maxsize=None)
def _get_scatter_kernel():
    mesh = plsc.VectorSubcoreMesh(core_axis_name="c", subcore_axis_name="s")

    @functools.partial(
        pl.kernel,
        out_type=jax.ShapeDtypeStruct((NC, N, MROW), jnp.float32),
        mesh=mesh,
        scratch_types=[
            pltpu.VMEM((S_CH,), jnp.int32),
            pltpu.VMEM((S_CH, MROW), jnp.float32),
            pltpu.VMEM_SHARED((N, MROW), jnp.float32),
            pltpu.SemaphoreType.DMA,
        ],
        compiler_params=pltpu.CompilerParams(use_tc_tiling_on_sc=False),
    )
    def _scatter_kernel(m, dst, zeros, out, idx_v, rows_v, accum, sem):
        c = lax.axis_index("c")
        s = lax.axis_index("s")
        w = s * NC + c
        base = w * PER_W

        @pl.when(s == 0)
        def _():
            pltpu.sync_copy(zeros, accum)

        plsc.subcore_barrier()

        @pl.loop(0, PER_W, step=S_CH)
        def _(off):
            e0 = base + off
            pltpu.sync_copy(dst.at[pl.ds(e0, S_CH)], idx_v)
            pltpu.sync_copy(m.at[pl.ds(e0, S_CH)], rows_v)
            pltpu.sync_copy(rows_v, accum.at[idx_v], add=True)

        plsc.subcore_barrier()

        @pl.when(s == 0)
        def _():
            pltpu.sync_copy(accum, out.at[c])

    return _scatter_kernel


NB = 2000


def _node_body(x, pos, p0, p1, w3a, w3b, b3, w4, b4, out_x, out_pos):
    agg = p0[...] + p1[...]
    am = agg[:, 0:MSG]
    ap = agg[:, MSG:MSG + 3]
    h1 = (jnp.dot(x[...], w3a[...], preferred_element_type=jnp.float32)
          + jnp.dot(am, w3b[...], preferred_element_type=jnp.float32)
          + b3[...])
    out_x[...] = (jnp.dot(_silu(h1), w4[...],
                          preferred_element_type=jnp.float32) + b4[...])
    out_pos[...] = pos[...] + ap


def _node_mlp(x, pos, p0, p1, w3a, w3b, b3, w4, b4):
    const = lambda shape: pl.BlockSpec(shape, lambda i: (0, 0))
    return pl.pallas_call(
        _node_body,
        grid=(N // NB,),
        in_specs=[
            pl.BlockSpec((NB, D), lambda i: (i, 0)),
            pl.BlockSpec((NB, 3), lambda i: (i, 0)),
            pl.BlockSpec((NB, MROW), lambda i: (i, 0)),
            pl.BlockSpec((NB, MROW), lambda i: (i, 0)),
            const((D, D)),
            const((MSG, D)),
            const((1, D)),
            const((D, D)),
            const((1, D)),
        ],
        out_specs=[
            pl.BlockSpec((NB, D), lambda i: (i, 0)),
            pl.BlockSpec((NB, 3), lambda i: (i, 0)),
        ],
        out_shape=[
            jax.ShapeDtypeStruct((N, D), jnp.float32),
            jax.ShapeDtypeStruct((N, 3), jnp.float32),
        ],
    )(x, pos, p0, p1, w3a, w3b, b3, w4, b4)


def kernel(x, pos, edge_index, W1, b1, W2, b2, W3, b3, W4, b4, W5, b5, W6, b6):
    src = edge_index[0].astype(jnp.int32)
    dst = edge_index[1].astype(jnp.int32)

    tp = jnp.pad(pos, ((0, 0), (0, PPAD - 3)))
    xu = jax.lax.bitcast_convert_type(
        x.astype(jnp.bfloat16), jnp.uint16).astype(jnp.uint32)
    txp = jax.lax.bitcast_convert_type(
        xu[:, :D // 2] | (xu[:, D // 2:] << 16), jnp.int32)

    gxp, gpd, gps = _get_gather_kernel()(txp, tp, dst, src)

    H = D // 2
    w1ab = jnp.concatenate(
        [W1[0:H], W1[D:D + H], W1[H:D], W1[D + H:2 * D]])
    w1c16 = jnp.tile(W1[2 * D:], (PPAD, 1))
    m = _edge_mlp(gxp, gpd, gps, w1ab, w1c16, b1[None, :],
                  W2.astype(jnp.bfloat16), b2[None, :], W5, b5[None, :],
                  W6, b6[None, :])

    partials = _get_scatter_kernel()(m, dst, jnp.zeros((N, MROW), jnp.float32))

    out_x, out_pos = _node_mlp(x, pos, partials[0], partials[1],
                               W3[:D], W3[D:], b3[None, :], W4, b4[None, :])
    return (out_x, out_pos)


# speedup vs baseline: 2.4761x; 1.0291x over previous
import functools

import jax
import jax.numpy as jnp
from jax import lax
from jax.experimental import pallas as pl
from jax.experimental.pallas import tpu as pltpu
from jax.experimental.pallas import tpu_sc as plsc

N = 10000
E = 320000
D = 128
MSG = 16
PPAD = 16
MROW = 32

NC = 2
NS = 16
NW = NC * NS
PER_W = E // NW
G_CH = 400
NIT = PER_W // G_CH
S_CH = 1000

def _silu(v):
    half = jnp.asarray(0.5, v.dtype)
    return v * (half * jnp.tanh(v * half) + half)


@functools.lru_cache(maxsize=None)
def _get_gather_kernel():
    mesh = plsc.VectorSubcoreMesh(core_axis_name="c", subcore_axis_name="s")

    @functools.partial(
        pl.kernel,
        out_type=(
            jax.ShapeDtypeStruct((E, D), jnp.int32),
            jax.ShapeDtypeStruct((E, PPAD), jnp.float32),
            jax.ShapeDtypeStruct((E, PPAD), jnp.float32),
        ),
        mesh=mesh,
        scratch_types=[
            pltpu.VMEM((2, G_CH), jnp.int32),
            pltpu.VMEM((2, G_CH), jnp.int32),
            pltpu.VMEM((2, G_CH, D // 2), jnp.int32),
            pltpu.VMEM((2, G_CH, D // 2), jnp.int32),
            pltpu.VMEM((G_CH, PPAD), jnp.float32),
            pltpu.VMEM((G_CH, PPAD), jnp.float32),
            pltpu.SemaphoreType.DMA,
            pltpu.SemaphoreType.DMA,
            pltpu.SemaphoreType.DMA,
            pltpu.SemaphoreType.DMA,
        ],
        compiler_params=pltpu.CompilerParams(use_tc_tiling_on_sc=False),
    )
    def _gather_kernel(tx, tp, dst, src, gx, gpd, gps,
                       idx_d, idx_s, xb_d, xb_s, pb_d, pb_s,
                       sem_a, sem_b, sem_i, sem_p):
        w = lax.axis_index("s") * NC + lax.axis_index("c")
        base = w * PER_W
        gsem = (sem_a, sem_b)

        def load_idx(i, buf):
            e0 = base + i * G_CH
            pltpu.async_copy(dst.at[pl.ds(e0, G_CH)], idx_d.at[buf], sem_i)
            pltpu.async_copy(src.at[pl.ds(e0, G_CH)], idx_s.at[buf], sem_i)

        def wait_idx(buf):
            pltpu.make_async_copy(dst.at[pl.ds(0, G_CH)], idx_d.at[buf],
                                  sem_i).wait()
            pltpu.make_async_copy(src.at[pl.ds(0, G_CH)], idx_s.at[buf],
                                  sem_i).wait()

        def fire_x(buf):
            pltpu.async_copy(tx.at[idx_d.at[buf]], xb_d.at[buf], gsem[buf])
            pltpu.async_copy(tx.at[idx_s.at[buf]], xb_s.at[buf], gsem[buf])

        def wait_x(buf):
            pltpu.make_async_copy(tx.at[idx_d.at[buf]], xb_d.at[buf],
                                  gsem[buf]).wait()
            pltpu.make_async_copy(tx.at[idx_s.at[buf]], xb_s.at[buf],
                                  gsem[buf]).wait()

        def fire_pos(buf):
            pltpu.async_copy(tp.at[idx_d.at[buf]], pb_d, sem_p)
            pltpu.async_copy(tp.at[idx_s.at[buf]], pb_s, sem_p)

        def wait_pos():
            pltpu.make_async_copy(tp.at[idx_d.at[0]], pb_d, sem_p).wait()
            pltpu.make_async_copy(tp.at[idx_s.at[0]], pb_s, sem_p).wait()

        def wb_x(i, buf):
            e0 = base + i * G_CH
            pltpu.sync_copy(xb_d.at[buf],
                            gx.at[pl.ds(e0, G_CH), pl.ds(0, D // 2)])
            pltpu.sync_copy(xb_s.at[buf],
                            gx.at[pl.ds(e0, G_CH), pl.ds(D // 2, D // 2)])

        def wb_pos(i):
            e0 = base + i * G_CH
            pltpu.sync_copy(pb_d, gpd.at[pl.ds(e0, G_CH)])
            pltpu.sync_copy(pb_s, gps.at[pl.ds(e0, G_CH)])

        load_idx(0, 0)
        wait_idx(0)
        fire_x(0)
        fire_pos(0)
        load_idx(1, 1)

        @pl.loop(0, (NIT - 1) // 2)
        def _(k):
            i1 = 2 * k + 1
            wait_idx(1)
            fire_x(1)
            wait_x(0)
            wb_x(i1 - 1, 0)
            wait_pos()
            wb_pos(i1 - 1)
            fire_pos(1)

            @pl.when(i1 + 1 < NIT)
            def _():
                load_idx(i1 + 1, 0)
                wait_idx(0)
                fire_x(0)

            wait_x(1)
            wb_x(i1, 1)
            wait_pos()
            wb_pos(i1)

            @pl.when(i1 + 1 < NIT)
            def _():
                fire_pos(0)

            @pl.when(i1 + 2 < NIT)
            def _():
                load_idx(i1 + 2, 1)

        wait_x(0)
        wb_x(NIT - 1, 0)
        wait_pos()
        wb_pos(NIT - 1)

    return _gather_kernel


EB = 4000


def _edge_body(gx, gpd, gps, w1ab, w1c16, b1, w2, b2, w5, b5, w6, b6, out):
    rel = gpd[...] - gps[...]
    rel2 = rel * rel
    w = gx[...]
    xlo = lax.bitcast_convert_type(w << 16, jnp.float32)
    xhi = lax.bitcast_convert_type(w & jnp.int32(-65536), jnp.float32)
    xcat = jnp.concatenate([xlo, xhi], axis=1)
    acc = lax.dot_general(xcat, w1ab[...], (((1,), (0,)), ((), ())),
                          preferred_element_type=jnp.float32)
    accd = lax.dot_general(rel2, w1c16[...], (((1,), (0,)), ((), ())),
                           preferred_element_type=jnp.float32)
    hpre = (acc + accd + b1[...]).astype(jnp.bfloat16)
    h = _silu(hpre)
    mpre = lax.dot_general(h, w2[...], (((1,), (0,)), ((), ())),
                           preferred_element_type=jnp.float32) + b2[...]
    msg = _silu(mpre)
    t = _silu(jnp.dot(msg, w5[...], preferred_element_type=jnp.float32)
              + b5[...])
    cw = jnp.dot(t, w6[...], preferred_element_type=jnp.float32) + b6[...]
    out[:, 0:MSG] = msg
    out[:, MSG:MROW] = rel * cw


def _edge_mlp(gx, gpd, gps, w1ab, w1c16, b1, w2, b2, w5, b5, w6, b6):
    ein = 2 * D + 1
    const = lambda shape: pl.BlockSpec(shape, lambda i: (0, 0))
    return pl.pallas_call(
        _edge_body,
        grid=(E // EB,),
        in_specs=[
            pl.BlockSpec((EB, D), lambda i: (i, 0)),
            pl.BlockSpec((EB, PPAD), lambda i: (i, 0)),
            pl.BlockSpec((EB, PPAD), lambda i: (i, 0)),
            const((2 * D, ein)),
            const((PPAD, ein)),
            const((1, ein)),
            const((ein, MSG)),
            const((1, MSG)),
            const((MSG, 2 * MSG)),
            const((1, 2 * MSG)),
            const((2 * MSG, 1)),
            const((1, 1)),
        ],
        out_specs=pl.BlockSpec((EB, MROW), lambda i: (i, 0)),
        out_shape=jax.ShapeDtypeStruct((E, MROW), jnp.float32),
    )(gx, gpd, gps, w1ab, w1c16, b1, w2, b2, w5, b5, w6, b6)


@functools.lru_cache(maxsize=None)
def _get_scatter_kernel():
    mesh = plsc.VectorSubcoreMesh(core_axis_name="c", subcore_axis_name="s")

    @functools.partial(
        pl.kernel,
        out_type=jax.ShapeDtypeStruct((NC, N, MROW), jnp.float32),
        mesh=mesh,
        scratch_types=[
            pltpu.VMEM((S_CH,), jnp.int32),
            pltpu.VMEM((S_CH, MROW), jnp.float32),
            pltpu.VMEM_SHARED((N, MROW), jnp.float32),
            pltpu.SemaphoreType.DMA,
        ],
        compiler_params=pltpu.CompilerParams(use_tc_tiling_on_sc=False),
    )
    def _scatter_kernel(m, dst, zeros, out, idx_v, rows_v, accum, sem):
        c = lax.axis_index("c")
        s = lax.axis_index("s")
        w = s * NC + c
        base = w * PER_W

        @pl.when(s == 0)
        def _():
            pltpu.sync_copy(zeros, accum)

        plsc.subcore_barrier()

        @pl.loop(0, PER_W, step=S_CH)
        def _(off):
            e0 = base + off
            pltpu.sync_copy(dst.at[pl.ds(e0, S_CH)], idx_v)
            pltpu.sync_copy(m.at[pl.ds(e0, S_CH)], rows_v)
            pltpu.sync_copy(rows_v, accum.at[idx_v], add=True)

        plsc.subcore_barrier()

        @pl.when(s == 0)
        def _():
            pltpu.sync_copy(accum, out.at[c])

    return _scatter_kernel


NB = 2000


def _node_body(x, pos, p0, p1, w3a, w3b, b3, w4, b4, out_x, out_pos):
    agg = p0[...] + p1[...]
    am = agg[:, 0:MSG]
    ap = agg[:, MSG:MSG + 3]
    h1 = (jnp.dot(x[...], w3a[...], preferred_element_type=jnp.float32)
          + jnp.dot(am, w3b[...], preferred_element_type=jnp.float32)
          + b3[...])
    out_x[...] = (jnp.dot(_silu(h1), w4[...],
                          preferred_element_type=jnp.float32) + b4[...])
    out_pos[...] = pos[...] + ap


def _node_mlp(x, pos, p0, p1, w3a, w3b, b3, w4, b4):
    const = lambda shape: pl.BlockSpec(shape, lambda i: (0, 0))
    return pl.pallas_call(
        _node_body,
        grid=(N // NB,),
        in_specs=[
            pl.BlockSpec((NB, D), lambda i: (i, 0)),
            pl.BlockSpec((NB, 3), lambda i: (i, 0)),
            pl.BlockSpec((NB, MROW), lambda i: (i, 0)),
            pl.BlockSpec((NB, MROW), lambda i: (i, 0)),
            const((D, D)),
            const((MSG, D)),
            const((1, D)),
            const((D, D)),
            const((1, D)),
        ],
        out_specs=[
            pl.BlockSpec((NB, D), lambda i: (i, 0)),
            pl.BlockSpec((NB, 3), lambda i: (i, 0)),
        ],
        out_shape=[
            jax.ShapeDtypeStruct((N, D), jnp.float32),
            jax.ShapeDtypeStruct((N, 3), jnp.float32),
        ],
    )(x, pos, p0, p1, w3a, w3b, b3, w4, b4)


def kernel(x, pos, edge_index, W1, b1, W2, b2, W3, b3, W4, b4, W5, b5, W6, b6):
    src = edge_index[0].astype(jnp.int32)
    dst = edge_index[1].astype(jnp.int32)

    tp = jnp.pad(pos, ((0, 0), (0, PPAD - 3)))
    xu = jax.lax.bitcast_convert_type(
        x.astype(jnp.bfloat16), jnp.uint16).astype(jnp.uint32)
    txp = jax.lax.bitcast_convert_type(
        xu[:, :D // 2] | (xu[:, D // 2:] << 16), jnp.int32)

    gxp, gpd, gps = _get_gather_kernel()(txp, tp, dst, src)

    H = D // 2
    w1ab = jnp.concatenate(
        [W1[0:H], W1[D:D + H], W1[H:D], W1[D + H:2 * D]])
    w1c16 = jnp.tile(W1[2 * D:], (PPAD, 1))
    m = _edge_mlp(gxp, gpd, gps, w1ab, w1c16, b1[None, :],
                  W2.astype(jnp.bfloat16), b2[None, :], W5, b5[None, :],
                  W6, b6[None, :])

    partials = _get_scatter_kernel()(m, dst, jnp.zeros((N, MROW), jnp.float32))

    out_x, out_pos = _node_mlp(x, pos, partials[0], partials[1],
                               W3[:D], W3[D:], b3[None, :], W4, b4[None, :])
    return (out_x, out_pos)
